# baseline (speedup 1.0000x reference)
import jax
import jax.numpy as jnp
from jax.experimental import pallas as pl

_BLOCK_ROWS = 8
_BLOCK_COLS = 28672


def _copy_body(in_ref, out_ref):
    out_ref[...] = in_ref[...]


def kernel(samples, sample_rate):
    del sample_rate
    rows, cols = samples.shape
    grid = (pl.cdiv(rows, _BLOCK_ROWS), pl.cdiv(cols, _BLOCK_COLS))
    return pl.pallas_call(
        _copy_body,
        out_shape=jax.ShapeDtypeStruct(samples.shape, samples.dtype),
        grid=grid,
        in_specs=[pl.BlockSpec((_BLOCK_ROWS, _BLOCK_COLS), lambda i, j: (i, j))],
        out_specs=pl.BlockSpec((_BLOCK_ROWS, _BLOCK_COLS), lambda i, j: (i, j)),
    )(samples)
